# Initial kernel scaffold; baseline (speedup 1.0000x reference)
import functools

import jax
import jax.numpy as jnp
from jax import lax
from jax.experimental import pallas as pl
from jax.experimental.pallas import tpu as pltpu
from jax.experimental.pallas import tpu_sc as plsc


def _fm_sc(B, F, V, D):
    info = plsc.get_sparse_core_info()
    NC, NS, L = info.num_cores, info.num_subcores, info.num_lanes
    NW = NC * NS
    assert D == L and B % NW == 0
    b_per_w = B // NW
    C = 128
    n_chunks = b_per_w // C
    CF = C * F

    mesh = plsc.VectorSubcoreMesh(core_axis_name="c", subcore_axis_name="s")

    @functools.partial(
        pl.kernel,
        mesh=mesh,
        out_type=jax.ShapeDtypeStruct((B,), jnp.float32),
        scratch_types=[
            pltpu.VMEM((CF,), jnp.int32),
            pltpu.VMEM((CF, D), jnp.float32),
            pltpu.VMEM((CF,), jnp.float32),
            pltpu.VMEM((C,), jnp.float32),
            pltpu.SemaphoreType.DMA,
        ],
    )
    def fm(table_hbm, idx_hbm, vals_hbm, out_hbm, idx_v, rows_v, vals_v, out_v, sem):
        wid = lax.axis_index("s") * NC + lax.axis_index("c")

        def chunk_body(j, carry):
            base_s = wid * b_per_w + j * C
            pltpu.sync_copy(idx_hbm.at[pl.ds(base_s * F, CF)], idx_v)
            pltpu.sync_copy(vals_hbm.at[pl.ds(base_s * F, CF)], vals_v)
            pltpu.async_copy(table_hbm.at[idx_v], rows_v, sem).wait()

            def sample_body(b, carry2):
                p0 = b * F
                acc = jnp.zeros((L,), jnp.float32)
                acc2 = jnp.zeros((L,), jnp.float32)
                for f in range(F):
                    row = rows_v[p0 + f, :]
                    vf = vals_v[p0 + f]
                    w = row * vf
                    acc = acc + w
                    acc2 = acc2 + w * w
                tot = jnp.sum(acc * acc - acc2)
                out_v[b] = 0.5 * tot
                return carry2

            lax.fori_loop(0, C, sample_body, 0)
            pltpu.sync_copy(out_v, out_hbm.at[pl.ds(base_s, C)])
            return carry

        lax.fori_loop(0, n_chunks, chunk_body, 0)

    return fm


def kernel(feature_indices, feature_values, embedding_weight):
    B, F = feature_indices.shape
    V, D = embedding_weight.shape
    idx_flat = feature_indices.reshape(B * F).astype(jnp.int32)
    vals_flat = feature_values.reshape(B * F)
    out = _fm_sc(B, F, V, D)(embedding_weight, idx_flat, vals_flat)
    return out.reshape(B, 1)


# speedup vs baseline: 1.1939x; 1.1939x over previous
import functools

import jax
import jax.numpy as jnp
from jax import lax
from jax.experimental import pallas as pl
from jax.experimental.pallas import tpu as pltpu
from jax.experimental.pallas import tpu_sc as plsc


def _fm_sc(B, F, V, D):
    info = plsc.get_sparse_core_info()
    NC, NS, L = info.num_cores, info.num_subcores, info.num_lanes
    NW = NC * NS
    assert D == L and B % NW == 0
    b_per_w = B // NW
    C = 128
    n_chunks = b_per_w // C
    CF = C * F
    FP = 32

    mesh = plsc.VectorSubcoreMesh(core_axis_name="c", subcore_axis_name="s")

    @functools.partial(
        pl.kernel,
        mesh=mesh,
        out_type=jax.ShapeDtypeStruct((B,), jnp.float32),
        compiler_params=pltpu.CompilerParams(
            needs_layout_passes=False, use_tc_tiling_on_sc=False
        ),
        scratch_types=[
            pltpu.VMEM((CF,), jnp.int32),
            pltpu.VMEM((CF, D), jnp.float32),
            pltpu.VMEM((C * FP,), jnp.float32),
            pltpu.VMEM((C, D), jnp.float32),
            pltpu.VMEM((C,), jnp.float32),
            pltpu.SemaphoreType.DMA,
        ],
    )
    def fm(table_hbm, idx_hbm, vals_hbm, out_hbm, idx_v, rows_v, vals_v, diffs_v, out_v, sem):
        wid = lax.axis_index("s") * NC + lax.axis_index("c")

        def chunk_body(j, carry):
            base_s = wid * b_per_w + j * C
            pltpu.sync_copy(idx_hbm.at[pl.ds(base_s * F, CF)], idx_v)
            pltpu.sync_copy(vals_hbm.at[pl.ds(base_s * FP, C * FP)], vals_v)
            pltpu.async_copy(table_hbm.at[idx_v], rows_v, sem).wait()

            lane = lax.iota(jnp.int32, L)

            def sample_body(b, carry2):
                p0 = b * F
                v0 = vals_v[pl.ds(b * FP, L)]
                v1 = vals_v[pl.ds(b * FP + L, L)]
                acc = jnp.zeros((L,), jnp.float32)
                acc2 = jnp.zeros((L,), jnp.float32)
                for f in range(F):
                    row = rows_v[p0 + f, :]
                    vf = v0[f] if f < L else v1[f - L]
                    w = row * vf
                    acc = acc + w
                    acc2 = acc2 + w * w
                diffs_v[b, :] = acc * acc - acc2
                return carry2

            lax.fori_loop(0, C, sample_body, 0)

            def group_body(g, carry2):
                row = g * L + lane
                tot = jnp.zeros((L,), jnp.float32)
                for d in range(D):
                    col = jnp.full((L,), d, jnp.int32)
                    tot = tot + plsc.load_gather(diffs_v, [row, col])
                out_v[pl.ds(g * L, L)] = 0.5 * tot
                return carry2

            lax.fori_loop(0, C // L, group_body, 0)
            pltpu.sync_copy(out_v, out_hbm.at[pl.ds(base_s, C)])
            return carry

        lax.fori_loop(0, n_chunks, chunk_body, 0)

    return fm


def kernel(feature_indices, feature_values, embedding_weight):
    B, F = feature_indices.shape
    V, D = embedding_weight.shape
    idx_flat = feature_indices.reshape(B * F).astype(jnp.int32)
    vals_pad = jnp.pad(feature_values, ((0, 0), (0, 32 - F))).reshape(B * 32)
    out = _fm_sc(B, F, V, D)(embedding_weight, idx_flat, vals_pad)
    return out.reshape(B, 1)
